# Initial kernel scaffold; baseline (speedup 1.0000x reference)
import jax
import jax.numpy as jnp
from jax.experimental import pallas as pl


def kernel(feature, edge_index, W, b, gamma, beta):
    raise NotImplementedError("write your pallas kernel here")


# speedup vs baseline: 16.9034x; 16.9034x over previous
import functools

import jax
import jax.numpy as jnp
from jax import lax
from jax.experimental import pallas as pl
from jax.experimental.pallas import tpu as pltpu
from jax.experimental.pallas import tpu_sc as plsc

N = 10000
E = 320000
D = 128
EPS = 1e-5

NC, NS, LANES = 2, 16, 16
NW = NC * NS
B = 128
K = -(-E // (NW * B))
EPT = K * B
EPAD = NW * EPT
NACC = ((N + 1 + 127) // 128) * 128
ROWS = NACC // NS
GW = 128

_mesh = plsc.VectorSubcoreMesh(core_axis_name="c", subcore_axis_name="s")


@functools.partial(
    pl.kernel,
    out_type=jax.ShapeDtypeStruct((NC, NACC, GW), jnp.float32),
    mesh=_mesh,
    scratch_types=[
        pltpu.VMEM((K, B), jnp.int32),
        pltpu.VMEM((B, GW), jnp.float32),
        pltpu.VMEM_SHARED((NACC, GW), jnp.float32),
    ],
)
def _sc_deg(dst_hbm, ones_hbm, zeros_hbm, degp_hbm, dst_v, ones_v, acc_sh):
    cid = lax.axis_index("c")
    sid = lax.axis_index("s")
    wid = cid * NS + sid
    pltpu.sync_copy(zeros_hbm.at[pl.ds(sid * ROWS, ROWS)],
                    acc_sh.at[pl.ds(sid * ROWS, ROWS)])
    pltpu.sync_copy(ones_hbm, ones_v)
    pltpu.sync_copy(dst_hbm.at[wid], dst_v)
    plsc.subcore_barrier()

    def body(j, c):
        pltpu.sync_copy(ones_v, acc_sh.at[dst_v.at[j]], add=True)
        return c

    lax.fori_loop(0, K, body, 0)
    plsc.subcore_barrier()
    pltpu.sync_copy(acc_sh.at[pl.ds(sid * ROWS, ROWS)],
                    degp_hbm.at[cid, pl.ds(sid * ROWS, ROWS)])


@functools.partial(
    pl.kernel,
    out_type=jax.ShapeDtypeStruct((NC, NACC, D), jnp.float32),
    mesh=_mesh,
    scratch_types=[
        pltpu.VMEM((K, B), jnp.int32),
        pltpu.VMEM((K, B), jnp.int32),
        pltpu.VMEM((B, D), jnp.float32),
        pltpu.VMEM_SHARED((NACC, D), jnp.float32),
        pltpu.SemaphoreType.DMA,
    ],
)
def _sc_msg(y_hbm, src_hbm, dst_hbm, zeros_hbm, accp_hbm,
            src_v, dst_v, gbuf, acc_sh, sem):
    cid = lax.axis_index("c")
    sid = lax.axis_index("s")
    wid = cid * NS + sid
    pltpu.sync_copy(zeros_hbm.at[pl.ds(sid * ROWS, ROWS)],
                    acc_sh.at[pl.ds(sid * ROWS, ROWS)])
    pltpu.sync_copy(src_hbm.at[wid], src_v)
    pltpu.sync_copy(dst_hbm.at[wid], dst_v)
    plsc.subcore_barrier()

    def body(j, c):
        pltpu.async_copy(y_hbm.at[src_v.at[j]], gbuf, sem).wait()
        pltpu.sync_copy(gbuf, acc_sh.at[dst_v.at[j]], add=True)
        return c

    lax.fori_loop(0, K, body, 0)
    plsc.subcore_barrier()
    pltpu.sync_copy(acc_sh.at[pl.ds(sid * ROWS, ROWS)],
                    accp_hbm.at[cid, pl.ds(sid * ROWS, ROWS)])


def _tc_prep_body(f_ref, w_ref, degp_ref, y_ref):
    deg = degp_ref[0, 0:N, 0:1] + degp_ref[1, 0:N, 0:1] + 1.0
    dis = lax.rsqrt(deg)
    x = jnp.dot(f_ref[...], w_ref[...],
                preferred_element_type=jnp.float32,
                precision=lax.Precision.HIGHEST)
    y_ref[...] = x * dis


_tc_prep = pl.pallas_call(
    _tc_prep_body,
    out_shape=jax.ShapeDtypeStruct((N, D), jnp.float32),
)


def _tc_fin_body(accp_ref, degp_ref, y_ref, b_ref, g_ref, be_ref, out_ref):
    deg = degp_ref[0, 0:N, 0:1] + degp_ref[1, 0:N, 0:1] + 1.0
    dis = lax.rsqrt(deg)
    t = (accp_ref[0, 0:N, :] + accp_ref[1, 0:N, :] + y_ref[...]) * dis
    t = t + b_ref[...]
    mean = jnp.mean(t, axis=0, keepdims=True)
    cen = t - mean
    var = jnp.mean(cen * cen, axis=0, keepdims=True)
    out_ref[...] = cen * lax.rsqrt(var + EPS) * g_ref[...] + be_ref[...]


_tc_fin = pl.pallas_call(
    _tc_fin_body,
    out_shape=jax.ShapeDtypeStruct((N, D), jnp.float32),
)


def kernel(feature, edge_index, W, b, gamma, beta):
    src = edge_index[0].astype(jnp.int32)
    dst = edge_index[1].astype(jnp.int32)
    pad = EPAD - E
    src_t = jnp.concatenate([src, jnp.zeros((pad,), jnp.int32)]).reshape(NW, K, B)
    dst_t = jnp.concatenate([dst, jnp.full((pad,), NACC - 1, jnp.int32)]
                            ).reshape(NW, K, B)
    ones_g = jnp.ones((B, GW), jnp.float32)
    zeros_g = jnp.zeros((NACC, GW), jnp.float32)
    zeros_d = jnp.zeros((NACC, D), jnp.float32)

    degp = _sc_deg(dst_t, ones_g, zeros_g)
    y = _tc_prep(feature, W, degp)
    accp = _sc_msg(y, src_t, dst_t, zeros_d)
    return _tc_fin(accp, degp, y, b.reshape(1, D), gamma.reshape(1, D),
                   beta.reshape(1, D))
